# Initial kernel scaffold; baseline (speedup 1.0000x reference)
import jax
import jax.numpy as jnp
from jax.experimental import pallas as pl


def kernel(hidden_states, W1, b1, W2, b2, Wr, br):
    raise NotImplementedError("write your pallas kernel here")


# speedup vs baseline: 1.1932x; 1.1932x over previous
import functools

import jax
import jax.numpy as jnp
import numpy as np
from jax import lax
from jax.experimental import pallas as pl
from jax.experimental.pallas import tpu as pltpu

D_MODEL = 2048
NUM_EXPERTS = 16
SPAN_SIZE = 32
OVERLAP = 8
STRIDE = SPAN_SIZE - OVERLAP


def _num_spans(seq_len):
    n = 0
    for start in range(0, seq_len, STRIDE):
        n += 1
        if start + SPAN_SIZE >= seq_len:
            break
    return n


GROUP = 5
GROUP_ROWS = 128


def _spans_agg_body(hs_ref, spans_ref, agg_ref, scratch, sem):
    b = pl.program_id(0)
    m = pl.program_id(1)
    copy = pltpu.make_async_copy(
        hs_ref.at[b, pl.ds(m * GROUP * STRIDE, GROUP_ROWS), :], scratch, sem)
    copy.start()
    copy.wait()
    blk = scratch[...].reshape(GROUP_ROWS // 8, 8, D_MODEL)
    gsum = jnp.sum(blk, axis=1)
    for k in range(GROUP):
        spans_ref[0, k] = scratch[pl.ds(k * STRIDE, SPAN_SIZE), :]
        s = gsum[3 * k] + gsum[3 * k + 1] + gsum[3 * k + 2] + gsum[3 * k + 3]
        agg_ref[0, k, 0] = s * (1.0 / SPAN_SIZE)


def _spans_and_agg(hidden_states, num_spans):
    B, S, D = hidden_states.shape
    n_groups = num_spans // GROUP
    spans, agg = pl.pallas_call(
        _spans_agg_body,
        grid=(B, n_groups),
        in_specs=[pl.BlockSpec(memory_space=pl.ANY)],
        out_specs=[
            pl.BlockSpec((1, GROUP, SPAN_SIZE, D), lambda b, m: (b, m, 0, 0)),
            pl.BlockSpec((1, GROUP, 1, D), lambda b, m: (b, m, 0, 0)),
        ],
        out_shape=[
            jax.ShapeDtypeStruct((B, num_spans, SPAN_SIZE, D), jnp.float32),
            jax.ShapeDtypeStruct((B, num_spans, 1, D), jnp.float32),
        ],
        scratch_shapes=[
            pltpu.VMEM((GROUP_ROWS, D), jnp.float32),
            pltpu.SemaphoreType.DMA,
        ],
    )(hidden_states)
    return spans, agg


def _mlp_body(n_valid, x_ref, w1_ref, b1_ref, w2_ref, b2_ref, wr_ref, br_ref,
              probs_ref, ids_ref, counts_ref, ent_ref):
    x = x_ref[...]
    h1 = jnp.dot(x, w1_ref[...], preferred_element_type=jnp.float32) + b1_ref[...]
    h = 0.5 * h1 * (1.0 + lax.erf(h1 * np.float32(1.0 / np.sqrt(2.0))))
    enc = jnp.dot(h, w2_ref[...], preferred_element_type=jnp.float32) + b2_ref[...]
    logits = jnp.dot(enc, wr_ref[...], preferred_element_type=jnp.float32) + br_ref[...]

    m = jnp.max(logits, axis=-1, keepdims=True)
    e = jnp.exp(logits - m)
    p = e / jnp.sum(e, axis=-1, keepdims=True)
    probs_ref[...] = p

    eidx = lax.broadcasted_iota(jnp.int32, logits.shape, 1)
    ids = jnp.min(jnp.where(logits == m, eidx, np.int32(NUM_EXPERTS)), axis=-1,
                  keepdims=True)
    ids_ref[...] = ids

    valid = lax.broadcasted_iota(jnp.int32, ids.shape, 0) < n_valid
    onehot = jnp.where(
        (ids == lax.broadcasted_iota(jnp.int32, (ids.shape[0], NUM_EXPERTS), 1))
        & valid, 1.0, 0.0)
    counts_ref[...] = jnp.sum(onehot, axis=0, keepdims=True)

    terms = -jnp.sum(p * jnp.log(p + 1e-10), axis=-1, keepdims=True)
    total = jnp.sum(jnp.where(valid, terms, 0.0))
    ent_ref[...] = jnp.full((1, 1), 1.0 / n_valid) * total


def _router(xp, W1, b1, W2, b2, Wr, br, n_valid):
    N = xp.shape[0]
    probs, ids, counts, ent = pl.pallas_call(
        functools.partial(_mlp_body, n_valid),
        out_shape=[
            jax.ShapeDtypeStruct((N, NUM_EXPERTS), jnp.float32),
            jax.ShapeDtypeStruct((N, 1), jnp.int32),
            jax.ShapeDtypeStruct((1, NUM_EXPERTS), jnp.float32),
            jax.ShapeDtypeStruct((1, 1), jnp.float32),
        ],
    )(xp, W1, b1.reshape(1, -1), W2, b2.reshape(1, -1), Wr, br.reshape(1, -1))
    return probs, ids, counts, ent


def kernel(hidden_states, W1, b1, W2, b2, Wr, br):
    B, S, D = hidden_states.shape
    num_spans = _num_spans(S)

    spans, agg = _spans_and_agg(hidden_states, num_spans)

    n_valid = B * num_spans
    n_pad = ((n_valid + 127) // 128) * 128
    xp = jnp.pad(agg.reshape(n_valid, D), ((0, n_pad - n_valid), (0, 0)))
    probs, ids, counts, ent = _router(xp, W1, b1, W2, b2, Wr, br, n_valid)

    routing_probs = probs[:n_valid].reshape(B, num_spans, NUM_EXPERTS)
    expert_ids = ids[:n_valid, 0].reshape(B, num_spans)
    expert_counts = counts[0]
    routing_entropy = ent[0, 0]
    return (spans, expert_ids, routing_probs, expert_counts, routing_entropy)
